# Initial kernel scaffold; baseline (speedup 1.0000x reference)
import functools

import jax
import jax.numpy as jnp
from jax import lax
from jax.experimental import pallas as pl
from jax.experimental.pallas import tpu as pltpu
from jax.experimental.pallas import tpu_sc as plsc

N_NODES = 10000
N_EDGES = 320000
D = 128

NC = 2
NT = 16
L = 128

EDGES_PER_TILE = -(-N_EDGES // (NT * L)) * L
CH = EDGES_PER_TILE // L
EP = EDGES_PER_TILE * NT

ACC_ROWS = ((N_NODES + NT - 1) // NT + 1) * NT
ZR = ACC_ROWS // NT
OR = N_NODES // NT


def _body(x_hbm, src_hbm, dst_hbm, zeros_hbm, out_hbm,
          acc, src_v, dst_v, rows_v, sem):
    c = lax.axis_index("c")
    s = lax.axis_index("s")

    pltpu.sync_copy(src_hbm.at[c, s], src_v)
    pltpu.sync_copy(dst_hbm.at[c, s], dst_v)

    pltpu.sync_copy(zeros_hbm, acc.at[pl.ds(s * ZR, ZR)])
    plsc.subcore_barrier()

    def chunk(j, carry):
        pltpu.async_copy(x_hbm.at[src_v.at[j]], rows_v, sem).wait()
        pltpu.sync_copy(rows_v, acc.at[dst_v.at[j]], add=True)
        return carry

    lax.fori_loop(0, CH, chunk, 0)
    plsc.subcore_barrier()

    pltpu.sync_copy(acc.at[pl.ds(s * OR, OR)],
                    out_hbm.at[c, pl.ds(s * OR, OR)])


@jax.jit
def _run(x, src_idx, dst_idx, zeros):
    mesh = plsc.VectorSubcoreMesh(core_axis_name="c", subcore_axis_name="s")
    f = pl.kernel(
        _body,
        out_type=jax.ShapeDtypeStruct((NC, N_NODES, D), jnp.float32),
        mesh=mesh,
        scratch_types=[
            pltpu.VMEM_SHARED((ACC_ROWS, D), jnp.float32),
            pltpu.VMEM((CH, L), jnp.int32),
            pltpu.VMEM((CH, L), jnp.int32),
            pltpu.VMEM((L, D), jnp.float32),
            pltpu.SemaphoreType.DMA,
        ],
    )
    return f(x, src_idx, dst_idx, zeros)


def _prep_indices(adj, dummy_row):
    dst = adj[0].astype(jnp.int32)
    src = adj[1].astype(jnp.int32)
    pad = EP - N_EDGES
    src = jnp.concatenate([src, jnp.zeros((pad,), jnp.int32)])
    dst = jnp.concatenate([dst, jnp.full((pad,), dummy_row, jnp.int32)])
    return src.reshape(NT, CH, L), dst.reshape(NT, CH, L)


def kernel(x, adj_t, adj_t2):
    s1, d1 = _prep_indices(adj_t, N_NODES)
    s2, d2 = _prep_indices(adj_t2, N_NODES)
    src_idx = jnp.stack([s1, s2])
    dst_idx = jnp.stack([d1, d2])
    zeros = jnp.zeros((ZR, D), jnp.float32)
    out = _run(x, src_idx, dst_idx, zeros)
    return jnp.concatenate([out[0], out[1]], axis=1)


# speedup vs baseline: 3.4873x; 3.4873x over previous
import functools

import jax
import jax.numpy as jnp
from jax import lax
from jax.experimental import pallas as pl
from jax.experimental.pallas import tpu as pltpu
from jax.experimental.pallas import tpu_sc as plsc

N_NODES = 10000
N_EDGES = 320000
D = 128

NC = 2
NT = 16
L = 128

IB = 32
NB = 5
CH = IB * NB
EDGES_PER_TILE = CH * L
EP = EDGES_PER_TILE * NT

ACC_ROWS = 10240
ZR = ACC_ROWS // NT


def _body(x_hbm, src_hbm, dst_hbm, zeros_hbm, out_hbm,
          acc, src_v, dst_v, rows_v, sem):
    c = lax.axis_index("c")
    s = lax.axis_index("s")

    pltpu.sync_copy(zeros_hbm, acc.at[pl.ds(s * ZR, ZR)])
    plsc.subcore_barrier()

    def block(b, carry):
        pltpu.sync_copy(src_hbm.at[c, s, pl.ds(b * IB, IB)], src_v)
        pltpu.sync_copy(dst_hbm.at[c, s, pl.ds(b * IB, IB)], dst_v)

        def chunk(j, inner):
            pltpu.async_copy(x_hbm.at[src_v.at[j]], rows_v, sem).wait()
            pltpu.sync_copy(rows_v, acc.at[dst_v.at[j]], add=True)
            return inner

        lax.fori_loop(0, IB, chunk, 0)
        return carry

    lax.fori_loop(0, NB, block, 0)
    plsc.subcore_barrier()

    pltpu.sync_copy(acc.at[pl.ds(s * ZR, ZR)],
                    out_hbm.at[c, pl.ds(s * ZR, ZR)])


@jax.jit
def _run(x, src_idx, dst_idx, zeros):
    mesh = plsc.VectorSubcoreMesh(core_axis_name="c", subcore_axis_name="s")
    f = pl.kernel(
        _body,
        out_type=jax.ShapeDtypeStruct((NC, ACC_ROWS, D), jnp.float32),
        mesh=mesh,
        scratch_types=[
            pltpu.VMEM_SHARED((ACC_ROWS, D), jnp.float32),
            pltpu.VMEM((IB, L), jnp.int32),
            pltpu.VMEM((IB, L), jnp.int32),
            pltpu.VMEM((L, D), jnp.float32),
            pltpu.SemaphoreType.DMA,
        ],
        compiler_params=pltpu.CompilerParams(use_tc_tiling_on_sc=False),
    )
    return f(x, src_idx, dst_idx, zeros)


def _prep_indices(adj, dummy_row):
    dst = adj[0].astype(jnp.int32)
    src = adj[1].astype(jnp.int32)
    pad = EP - N_EDGES
    src = jnp.concatenate([src, jnp.zeros((pad,), jnp.int32)])
    dst = jnp.concatenate([dst, jnp.full((pad,), dummy_row, jnp.int32)])
    return src.reshape(NT, CH, L), dst.reshape(NT, CH, L)


def kernel(x, adj_t, adj_t2):
    s1, d1 = _prep_indices(adj_t, N_NODES)
    s2, d2 = _prep_indices(adj_t2, N_NODES)
    src_idx = jnp.stack([s1, s2])
    dst_idx = jnp.stack([d1, d2])
    zeros = jnp.zeros((ZR, D), jnp.float32)
    out = _run(x, src_idx, dst_idx, zeros)
    return jnp.concatenate([out[0, :N_NODES], out[1, :N_NODES]], axis=1)
